# baseline (speedup 1.0000x reference)
import functools

import jax
import jax.numpy as jnp
from jax import lax
from jax.experimental import pallas as pl
from jax.experimental.pallas import tpu as pltpu
from jax.experimental.pallas import tpu_sc as plsc

D_MODEL = 1024
N_ROWS = 16384
_L = 16
_NC, _NS = 2, 16
_NW = _NC * _NS
_B_PER_W = N_ROWS // _NW
_C = 16
_CHUNKS = _B_PER_W // _C


def _sc_body(pe_hbm, idx_hbm, x_hbm, out_hbm, idx_v, x_v, pe_v, sem):
    wid = lax.axis_index("s") * _NC + lax.axis_index("c")
    pltpu.sync_copy(idx_hbm.at[wid], idx_v)

    def chunk_body(j, carry):
        base = wid * _B_PER_W + j * _C
        pltpu.sync_copy(x_hbm.at[pl.ds(base, _C)], x_v)
        pltpu.async_copy(pe_hbm.at[idx_v.at[j]], pe_v, sem).wait()

        def row_body(r, c2):
            for k in range(D_MODEL // _L):
                sl = pl.ds(k * _L, _L)
                x_v[r, sl] = x_v[r, sl] + pe_v[r, sl]
            return c2

        lax.fori_loop(0, _C, row_body, 0)
        pltpu.sync_copy(x_v, out_hbm.at[pl.ds(base, _C)])
        return carry

    lax.fori_loop(0, _CHUNKS, chunk_body, 0)


@jax.jit
def _pe_add(pe, idx3, x2):
    f = functools.partial(
        pl.kernel,
        mesh=plsc.VectorSubcoreMesh(core_axis_name="c", subcore_axis_name="s"),
        out_type=jax.ShapeDtypeStruct((N_ROWS, D_MODEL), jnp.float32),
        scratch_types=[
            pltpu.VMEM((_CHUNKS, _C), jnp.int32),
            pltpu.VMEM((_C, D_MODEL), jnp.float32),
            pltpu.VMEM((_C, D_MODEL), jnp.float32),
            pltpu.SemaphoreType.DMA,
        ],
    )(_sc_body)
    return f(pe, idx3, x2)


def kernel(x, step, pe):
    s, b, d = x.shape
    x2 = x.reshape(s * b, d)
    idx3 = step.reshape(_NW, _CHUNKS, _C).astype(jnp.int32)
    out = _pe_add(pe, idx3, x2)
    return out.reshape(s, b, d)


# speedup vs baseline: 1.3876x; 1.3876x over previous
import functools

import jax
import jax.numpy as jnp
from jax import lax
from jax.experimental import pallas as pl
from jax.experimental.pallas import tpu as pltpu
from jax.experimental.pallas import tpu_sc as plsc

D_MODEL = 1024
N_ROWS = 16384
_L = 16
_NC, _NS = 2, 16
_NW = _NC * _NS
_B_PER_W = N_ROWS // _NW
_C = 16
_CHUNKS = _B_PER_W // _C
_NPAIR = _CHUNKS // 2


def _sc_body(pe_hbm, idx_hbm, x_hbm, out_hbm, idx_v,
             x_v0, x_v1, pe_v0, pe_v1, o_v0, o_v1,
             sx0, sx1, sg0, sg1, so0, so1):
    wid = lax.axis_index("s") * _NC + lax.axis_index("c")
    base_w = wid * _B_PER_W
    pltpu.sync_copy(idx_hbm.at[wid], idx_v)

    xs = (x_v0, x_v1)
    pes = (pe_v0, pe_v1)
    outs = (o_v0, o_v1)
    sxs = (sx0, sx1)
    sgs = (sg0, sg1)
    sos = (so0, so1)

    def issue_loads(c, s):
        pltpu.async_copy(x_hbm.at[pl.ds(base_w + c * _C, _C)], xs[s], sxs[s])
        pltpu.async_copy(pe_hbm.at[idx_v.at[c]], pes[s], sgs[s])

    issue_loads(0, 0)
    issue_loads(1, 1)

    def pair_body(j2, carry):
        for s in (0, 1):
            c = j2 * 2 + s
            pltpu.make_async_copy(
                x_hbm.at[pl.ds(0, _C)], xs[s], sxs[s]).wait()
            pltpu.make_async_copy(
                pe_hbm.at[idx_v.at[0]], pes[s], sgs[s]).wait()

            @pl.when(j2 >= 1)
            def _():
                pltpu.make_async_copy(
                    outs[s], out_hbm.at[pl.ds(0, _C)], sos[s]).wait()

            def row_body(r, c2):
                for k in range(D_MODEL // _L):
                    sl = pl.ds(k * _L, _L)
                    outs[s][r, sl] = xs[s][r, sl] + pes[s][r, sl]
                return c2

            lax.fori_loop(0, _C, row_body, 0)

            pltpu.async_copy(
                outs[s], out_hbm.at[pl.ds(base_w + c * _C, _C)], sos[s])

            @pl.when(j2 < _NPAIR - 1)
            def _():
                issue_loads(c + 2, s)
        return carry

    lax.fori_loop(0, _NPAIR, pair_body, 0)

    for s in (0, 1):
        pltpu.make_async_copy(
            outs[s], out_hbm.at[pl.ds(0, _C)], sos[s]).wait()


@jax.jit
def _pe_add(pe, idx3, x2):
    f = functools.partial(
        pl.kernel,
        mesh=plsc.VectorSubcoreMesh(core_axis_name="c", subcore_axis_name="s"),
        out_type=jax.ShapeDtypeStruct((N_ROWS, D_MODEL), jnp.float32),
        scratch_types=[
            pltpu.VMEM((_CHUNKS, _C), jnp.int32),
            pltpu.VMEM((_C, D_MODEL), jnp.float32),
            pltpu.VMEM((_C, D_MODEL), jnp.float32),
            pltpu.VMEM((_C, D_MODEL), jnp.float32),
            pltpu.VMEM((_C, D_MODEL), jnp.float32),
            pltpu.VMEM((_C, D_MODEL), jnp.float32),
            pltpu.VMEM((_C, D_MODEL), jnp.float32),
            pltpu.SemaphoreType.DMA,
            pltpu.SemaphoreType.DMA,
            pltpu.SemaphoreType.DMA,
            pltpu.SemaphoreType.DMA,
            pltpu.SemaphoreType.DMA,
            pltpu.SemaphoreType.DMA,
        ],
    )(_sc_body)
    return f(pe, idx3, x2)


def kernel(x, step, pe):
    s, b, d = x.shape
    x2 = x.reshape(s * b, d)
    idx3 = step.reshape(_NW, _CHUNKS, _C).astype(jnp.int32)
    out = _pe_add(pe, idx3, x2)
    return out.reshape(s, b, d)


# speedup vs baseline: 1.8488x; 1.3324x over previous
import functools

import jax
import jax.numpy as jnp
from jax import lax
from jax.experimental import pallas as pl
from jax.experimental.pallas import tpu as pltpu
from jax.experimental.pallas import tpu_sc as plsc

SEQ = 4096
BATCH = 4
D_MODEL = 1024
_L = 16
_NC, _NS = 2, 16
_NW = _NC * _NS
_S_PER_W = SEQ // _NW
_CS = 4
_C = _CS * BATCH
_CHUNKS = _S_PER_W // _CS
_NPAIR = _CHUNKS // 2


def _sc_body(pe_hbm, idx_hbm, x_hbm, out_hbm, idx_v,
             x_v0, x_v1, pe_v0, pe_v1, o_v0, o_v1,
             sx0, sx1, sg0, sg1, so0, so1):
    wid = lax.axis_index("s") * _NC + lax.axis_index("c")
    sbase_w = wid * _S_PER_W
    pltpu.sync_copy(idx_hbm.at[wid], idx_v)

    xs = (x_v0, x_v1)
    pes = (pe_v0, pe_v1)
    outs = (o_v0, o_v1)
    sxs = (sx0, sx1)
    sgs = (sg0, sg1)
    sos = (so0, so1)

    def issue_loads(c, s):
        pltpu.async_copy(
            x_hbm.at[pl.ds(sbase_w + c * _CS, _CS)], xs[s], sxs[s])
        pltpu.async_copy(pe_hbm.at[idx_v.at[c]], pes[s], sgs[s])

    issue_loads(0, 0)
    issue_loads(1, 1)

    def pair_body(j2, carry):
        for s in (0, 1):
            c = j2 * 2 + s
            pltpu.make_async_copy(
                x_hbm.at[pl.ds(0, _CS)], xs[s], sxs[s]).wait()
            pltpu.make_async_copy(
                pe_hbm.at[idx_v.at[0]], pes[s], sgs[s]).wait()

            @pl.when(j2 >= 1)
            def _():
                pltpu.make_async_copy(
                    outs[s], out_hbm.at[pl.ds(0, _CS)], sos[s]).wait()

            def row_body(r, c2):
                a = r // BATCH
                b = r % BATCH
                for k in range(D_MODEL // _L):
                    sl = pl.ds(k * _L, _L)
                    outs[s][a, b, sl] = xs[s][a, b, sl] + pes[s][r, sl]
                return c2

            lax.fori_loop(0, _C, row_body, 0)

            pltpu.async_copy(
                outs[s], out_hbm.at[pl.ds(sbase_w + c * _CS, _CS)], sos[s])

            @pl.when(j2 < _NPAIR - 1)
            def _():
                issue_loads(c + 2, s)
        return carry

    lax.fori_loop(0, _NPAIR, pair_body, 0)

    for s in (0, 1):
        pltpu.make_async_copy(
            outs[s], out_hbm.at[pl.ds(0, _CS)], sos[s]).wait()


@jax.jit
def _pe_add(pe, idx3, x):
    f = functools.partial(
        pl.kernel,
        mesh=plsc.VectorSubcoreMesh(core_axis_name="c", subcore_axis_name="s"),
        out_type=jax.ShapeDtypeStruct((SEQ, BATCH, D_MODEL), jnp.float32),
        scratch_types=[
            pltpu.VMEM((_CHUNKS, _C), jnp.int32),
            pltpu.VMEM((_CS, BATCH, D_MODEL), jnp.float32),
            pltpu.VMEM((_CS, BATCH, D_MODEL), jnp.float32),
            pltpu.VMEM((_C, D_MODEL), jnp.float32),
            pltpu.VMEM((_C, D_MODEL), jnp.float32),
            pltpu.VMEM((_CS, BATCH, D_MODEL), jnp.float32),
            pltpu.VMEM((_CS, BATCH, D_MODEL), jnp.float32),
            pltpu.SemaphoreType.DMA,
            pltpu.SemaphoreType.DMA,
            pltpu.SemaphoreType.DMA,
            pltpu.SemaphoreType.DMA,
            pltpu.SemaphoreType.DMA,
            pltpu.SemaphoreType.DMA,
        ],
    )(_sc_body)
    return f(pe, idx3, x)


def kernel(x, step, pe):
    idx3 = step.reshape(_NW, _CHUNKS, _C).astype(jnp.int32)
    return _pe_add(pe, idx3, x)
